# Initial kernel scaffold; baseline (speedup 1.0000x reference)
import jax
import jax.numpy as jnp
from jax.experimental import pallas as pl


def kernel(x, Wg, w1, w2, w3, ws1, ws2):
    raise NotImplementedError("write your pallas kernel here")


# speedup vs baseline: 1.3076x; 1.3076x over previous
import functools

import jax
import jax.numpy as jnp
from jax.experimental import pallas as pl
from jax.experimental.pallas import tpu as pltpu

DIM = 2048
INTER = 1408
E = 8
K = 2
T = 2048
TT = 4
TM = T // TT


def _moe_dense_body(x_ref, wg_ref, w1_ref, w3_ref, w2_ref, y_ref, gate_ref):
    e = pl.program_id(1)
    eidx = jax.lax.broadcasted_iota(jnp.int32, (TM, E), 1)

    @pl.when(e == 0)
    def _():
        xl = x_ref[...]
        logits = jnp.dot(xl, wg_ref[...].T, preferred_element_type=jnp.float32)
        m1 = jnp.max(logits, axis=1, keepdims=True)
        e1 = jnp.min(jnp.where(logits >= m1, eidx, E), axis=1, keepdims=True)
        l2 = jnp.where(eidx == e1, -1e30, logits)
        m2 = jnp.max(l2, axis=1, keepdims=True)
        e2 = jnp.min(jnp.where(l2 >= m2, eidx, E), axis=1, keepdims=True)
        q = jnp.exp(m2 - m1)
        wa = 1.0 / (1.0 + q)
        wb = 1.0 - wa
        gate_ref[...] = (jnp.where(eidx == e1, wa, 0.0)
                         + jnp.where(eidx == e2, wb, 0.0))
        y_ref[...] = jnp.zeros_like(y_ref)

    xb = x_ref[...].astype(jnp.bfloat16)
    a = jnp.dot(xb, w1_ref[0].T, preferred_element_type=jnp.float32)
    b = jnp.dot(xb, w3_ref[0].T, preferred_element_type=jnp.float32)
    h = (jax.nn.silu(a) * b).astype(jnp.bfloat16)
    out = jnp.dot(h, w2_ref[0].T, preferred_element_type=jnp.float32)
    gate_col = jnp.sum(jnp.where(eidx == e, gate_ref[...], 0.0), axis=1,
                       keepdims=True)
    y_ref[...] += gate_col * out


def _shared_body(y_ref, xb_ref, ws1_ref, ws2_ref, o_ref):
    a = jnp.dot(xb_ref[...], ws1_ref[...].T, preferred_element_type=jnp.float32)
    g = (0.5 * a * (1.0 + jax.lax.erf(a * 0.7071067811865476))).astype(
        jnp.bfloat16)
    o_ref[...] = y_ref[...] + jnp.dot(g, ws2_ref[...].T,
                                      preferred_element_type=jnp.float32)


@jax.jit
def kernel(x, Wg, w1, w2, w3, ws1, ws2):
    orig_shape = x.shape
    xf = x.reshape(-1, orig_shape[-1])
    w1b = w1.astype(jnp.bfloat16)
    w2b = w2.astype(jnp.bfloat16)
    w3b = w3.astype(jnp.bfloat16)
    ws1b = ws1.astype(jnp.bfloat16)
    ws2b = ws2.astype(jnp.bfloat16)

    y_moe = pl.pallas_call(
        _moe_dense_body,
        grid=(TT, E),
        in_specs=[
            pl.BlockSpec((TM, DIM), lambda t, e: (t, 0)),
            pl.BlockSpec((E, DIM), lambda t, e: (0, 0)),
            pl.BlockSpec((1, INTER, DIM), lambda t, e: (e, 0, 0)),
            pl.BlockSpec((1, INTER, DIM), lambda t, e: (e, 0, 0)),
            pl.BlockSpec((1, DIM, INTER), lambda t, e: (e, 0, 0)),
        ],
        out_specs=pl.BlockSpec((TM, DIM), lambda t, e: (t, 0)),
        out_shape=jax.ShapeDtypeStruct((T, DIM), jnp.float32),
        scratch_shapes=[pltpu.VMEM((TM, E), jnp.float32)],
    )(xf, Wg, w1b, w3b, w2b)

    xb = xf.astype(jnp.bfloat16)
    y = pl.pallas_call(
        _shared_body,
        grid=(TT,),
        in_specs=[
            pl.BlockSpec((TM, DIM), lambda t: (t, 0)),
            pl.BlockSpec((TM, DIM), lambda t: (t, 0)),
            pl.BlockSpec((INTER, DIM), lambda t: (0, 0)),
            pl.BlockSpec((DIM, INTER), lambda t: (0, 0)),
        ],
        out_specs=pl.BlockSpec((TM, DIM), lambda t: (t, 0)),
        out_shape=jax.ShapeDtypeStruct((T, DIM), jnp.float32),
    )(y_moe, xb, ws1b, ws2b)

    return y.reshape(orig_shape)


# speedup vs baseline: 1.3098x; 1.0017x over previous
import functools

import jax
import jax.numpy as jnp
from jax.experimental import pallas as pl
from jax.experimental.pallas import tpu as pltpu

DIM = 2048
INTER = 1408
E = 8
K = 2
T = 2048
TT = 4
TM = T // TT


def _moe_dense_body(x_ref, wg_ref, w1_ref, w3_ref, w2_ref, y_ref, gate_ref):
    e = pl.program_id(1)
    eidx = jax.lax.broadcasted_iota(jnp.int32, (TM, E), 1)

    @pl.when(e == 0)
    def _():
        xl = x_ref[...]
        logits = jnp.dot(xl, wg_ref[...].T, preferred_element_type=jnp.float32)
        m1 = jnp.max(logits, axis=1, keepdims=True)
        e1 = jnp.min(jnp.where(logits >= m1, eidx, E), axis=1, keepdims=True)
        l2 = jnp.where(eidx == e1, -1e30, logits)
        m2 = jnp.max(l2, axis=1, keepdims=True)
        e2 = jnp.min(jnp.where(l2 >= m2, eidx, E), axis=1, keepdims=True)
        q = jnp.exp(m2 - m1)
        wa = 1.0 / (1.0 + q)
        wb = 1.0 - wa
        gate_ref[...] = (jnp.where(eidx == e1, wa, 0.0)
                         + jnp.where(eidx == e2, wb, 0.0))
        y_ref[...] = jnp.zeros_like(y_ref)

    xb = x_ref[...].astype(jnp.bfloat16)
    a = jnp.dot(xb, w1_ref[0].T, preferred_element_type=jnp.float32)
    b = jnp.dot(xb, w3_ref[0].T, preferred_element_type=jnp.float32)
    h = (jax.nn.silu(a) * b).astype(jnp.bfloat16)
    out = jnp.dot(h, w2_ref[0].T, preferred_element_type=jnp.float32)
    gate_col = jnp.sum(jnp.where(eidx == e, gate_ref[...], 0.0), axis=1,
                       keepdims=True)
    y_ref[...] += gate_col * out


def _shared_body(y_ref, xb_ref, ws1_ref, ws2_ref, o_ref):
    a = jnp.dot(xb_ref[...], ws1_ref[...].T, preferred_element_type=jnp.float32)
    g = (0.5 * a * (1.0 + jax.lax.erf(a * 0.7071067811865476))).astype(
        jnp.bfloat16)
    o_ref[...] = y_ref[...] + jnp.dot(g, ws2_ref[...].T,
                                      preferred_element_type=jnp.float32)


@jax.jit
def kernel(x, Wg, w1, w2, w3, ws1, ws2):
    orig_shape = x.shape
    xf = x.reshape(-1, orig_shape[-1])
    w1b = w1.astype(jnp.bfloat16)
    w2b = w2.astype(jnp.bfloat16)
    w3b = w3.astype(jnp.bfloat16)
    ws1b = ws1.astype(jnp.bfloat16)
    ws2b = ws2.astype(jnp.bfloat16)

    y_moe = pl.pallas_call(
        _moe_dense_body,
        grid=(TT, E),
        in_specs=[
            pl.BlockSpec((TM, DIM), lambda t, e: (t, 0)),
            pl.BlockSpec((E, DIM), lambda t, e: (0, 0)),
            pl.BlockSpec((1, INTER, DIM), lambda t, e: (e, 0, 0)),
            pl.BlockSpec((1, INTER, DIM), lambda t, e: (e, 0, 0)),
            pl.BlockSpec((1, DIM, INTER), lambda t, e: (e, 0, 0)),
        ],
        out_specs=pl.BlockSpec((TM, DIM), lambda t, e: (t, 0)),
        out_shape=jax.ShapeDtypeStruct((T, DIM), jnp.float32),
        scratch_shapes=[pltpu.VMEM((TM, E), jnp.float32)],
        compiler_params=pltpu.CompilerParams(
            dimension_semantics=("parallel", "arbitrary")),
    )(xf, Wg, w1b, w3b, w2b)

    xb = xf.astype(jnp.bfloat16)
    y = pl.pallas_call(
        _shared_body,
        grid=(TT,),
        in_specs=[
            pl.BlockSpec((TM, DIM), lambda t: (t, 0)),
            pl.BlockSpec((TM, DIM), lambda t: (t, 0)),
            pl.BlockSpec((INTER, DIM), lambda t: (0, 0)),
            pl.BlockSpec((DIM, INTER), lambda t: (0, 0)),
        ],
        out_specs=pl.BlockSpec((TM, DIM), lambda t: (t, 0)),
        out_shape=jax.ShapeDtypeStruct((T, DIM), jnp.float32),
        compiler_params=pltpu.CompilerParams(
            dimension_semantics=("parallel",)),
    )(y_moe, xb, ws1b, ws2b)

    return y.reshape(orig_shape)
